# baseline (speedup 1.0000x reference)
import functools

import jax
import jax.numpy as jnp
import numpy as np
from jax import lax
from jax.experimental import pallas as pl
from jax.experimental.pallas import tpu as pltpu
from jax.experimental.pallas import tpu_sc as plsc

_B = 16384
_S = 26
_E = 16
_F = 4823
_NW = 32
_NSL = 2
_BSL = _B // _NSL
_NPW = _BSL * _S // _NW
_CHUNK = 1664
_NCH = _NPW // _CHUNK
_TC_BB = 1024


def _sc_gather(idx_flat, emb1_flat, emb2):
    mesh = plsc.VectorSubcoreMesh(core_axis_name="c", subcore_axis_name="s")

    @functools.partial(
        pl.kernel,
        mesh=mesh,
        out_type=[
            jax.ShapeDtypeStruct((_BSL * _S, _E), jnp.float32),
            jax.ShapeDtypeStruct((_BSL * _S,), jnp.float32),
        ],
        scratch_types=[
            pltpu.VMEM((_NPW,), jnp.int32),
            [pltpu.VMEM((_CHUNK, _E), jnp.float32)] * _NCH,
            [pltpu.VMEM((_CHUNK,), jnp.float32)] * _NCH,
            [pltpu.SemaphoreType.DMA] * (4 * _NCH),
        ],
        compiler_params=pltpu.CompilerParams(use_tc_tiling_on_sc=False),
    )
    def k(idx_hbm, emb1_hbm, emb2_hbm, rows_out, vals1_out,
          idx_v, rows_v, vals_v, sems):
        wid = lax.axis_index("s") * 2 + lax.axis_index("c")
        base = wid * _NPW
        pltpu.sync_copy(idx_hbm.at[pl.ds(base, _NPW)], idx_v)
        sgr = sems[0:_NCH]
        sgv = sems[_NCH:2 * _NCH]
        swr = sems[2 * _NCH:3 * _NCH]
        swv = sems[3 * _NCH:4 * _NCH]

        def idx_c(c):
            return idx_v.at[pl.ds(c * _CHUNK, _CHUNK)]

        def out_sl(c):
            return pl.ds(base + c * _CHUNK, _CHUNK)

        gr = [pltpu.async_copy(emb2_hbm.at[idx_c(c)], rows_v[c], sgr[c])
              for c in range(_NCH)]
        gv = [pltpu.async_copy(emb1_hbm.at[idx_c(c)], vals_v[c], sgv[c])
              for c in range(_NCH)]
        wr = [None] * _NCH
        wv = [None] * _NCH
        for c in range(_NCH):
            gr[c].wait()
            wr[c] = pltpu.async_copy(rows_v[c], rows_out.at[out_sl(c)],
                                     swr[c])
            gv[c].wait()
            wv[c] = pltpu.async_copy(vals_v[c], vals1_out.at[out_sl(c)],
                                     swv[c])
        for c in range(_NCH):
            wr[c].wait()
            wv[c].wait()

    return k(idx_flat, emb1_flat, emb2)


def _tc_body(e_ref, d_ref, e1_ref, fmw_ref, w1t_ref, w2t_ref, w3_ref,
             sel_ref, o_ref):
    e = e_ref[...]
    h1 = jnp.maximum(
        jnp.dot(e, w1t_ref[...], preferred_element_type=jnp.float32), 0.0)
    h2 = jnp.maximum(
        jnp.dot(h1.astype(jnp.bfloat16), w2t_ref[...],
                preferred_element_type=jnp.float32), 0.0)
    y3 = jnp.sum(h2 * w3_ref[...], axis=1, keepdims=True)
    ssum = jnp.dot(e, sel_ref[...], preferred_element_type=jnp.float32)
    ef = e.astype(jnp.float32)
    y2 = 0.5 * (jnp.sum(ssum * ssum, axis=1, keepdims=True)
                - jnp.sum(ef * ef, axis=1, keepdims=True))
    y1 = (jnp.sum(d_ref[...] * fmw_ref[...], axis=1, keepdims=True)
          + jnp.sum(e1_ref[...], axis=1, keepdims=True))
    z = y1 + y2 + y3
    o_ref[...] = 1.0 / (1.0 + jnp.exp(-z))


def _tc_mlp(e_flat, dense, e1, fm_w, w1t, w2t, w3, sel):
    nb = e_flat.shape[0]
    return pl.pallas_call(
        _tc_body,
        grid=(nb // _TC_BB,),
        in_specs=[
            pl.BlockSpec((_TC_BB, _S * _E), lambda i: (i, 0)),
            pl.BlockSpec((_TC_BB, 13), lambda i: (i, 0)),
            pl.BlockSpec((_TC_BB, _S), lambda i: (i, 0)),
            pl.BlockSpec((1, 13), lambda i: (0, 0)),
            pl.BlockSpec((_S * _E, 256), lambda i: (0, 0)),
            pl.BlockSpec((256, 256), lambda i: (0, 0)),
            pl.BlockSpec((1, 256), lambda i: (0, 0)),
            pl.BlockSpec((_S * _E, _E), lambda i: (0, 0)),
        ],
        out_specs=pl.BlockSpec((_TC_BB, 1), lambda i: (i, 0)),
        out_shape=jax.ShapeDtypeStruct((nb, 1), jnp.float32),
    )(e_flat, dense, e1, fm_w, w1t, w2t, w3, sel)


_SEL = np.tile(np.eye(_E, dtype=np.float32), (_S, 1))


def kernel(dense_input, sparse_input, emb1, emb2, fm_w, w1, w2, w3):
    emb1_flat = emb1.reshape(_F)
    w1t = w1.T.astype(jnp.bfloat16)
    w2t = w2.T.astype(jnp.bfloat16)
    sel = jnp.asarray(_SEL.astype(np.dtype(jnp.bfloat16)))
    outs = []
    for sl in range(_NSL):
        idx_sl = lax.slice_in_dim(sparse_input, sl * _BSL,
                                  (sl + 1) * _BSL).astype(jnp.int32).reshape(-1)
        rows, vals1 = _sc_gather(idx_sl, emb1_flat, emb2)
        e_flat = rows.reshape(_BSL, _S * _E).astype(jnp.bfloat16)
        e1 = vals1.reshape(_BSL, _S)
        d_sl = lax.slice_in_dim(dense_input, sl * _BSL, (sl + 1) * _BSL)
        outs.append(_tc_mlp(e_flat, d_sl, e1, fm_w, w1t, w2t, w3, sel))
    return jnp.concatenate(outs, axis=0)


# speedup vs baseline: 1.9722x; 1.9722x over previous
import functools

import jax
import jax.numpy as jnp
import numpy as np
from jax import lax
from jax.experimental import pallas as pl
from jax.experimental.pallas import tpu as pltpu
from jax.experimental.pallas import tpu_sc as plsc

_B = 16384
_S = 26
_E = 16
_F = 4823
_NW = 32
_SLICES = (6144, 6144, 4096)
_CHUNK = 1664
_TC_BB = 1024


@functools.lru_cache(maxsize=None)
def _sc_gather_fn(bsl):
    mesh = plsc.VectorSubcoreMesh(core_axis_name="c", subcore_axis_name="s")
    npw = bsl * _S // _NW
    nch = npw // _CHUNK

    @functools.partial(
        pl.kernel,
        mesh=mesh,
        out_type=[
            jax.ShapeDtypeStruct((bsl * _S, _E), jnp.float32),
            jax.ShapeDtypeStruct((bsl * _S,), jnp.float32),
        ],
        scratch_types=[
            pltpu.VMEM((npw,), jnp.int32),
            [pltpu.VMEM((_CHUNK, _E), jnp.float32)] * nch,
            [pltpu.VMEM((_CHUNK,), jnp.float32)] * nch,
            [pltpu.SemaphoreType.DMA] * (4 * nch),
        ],
        compiler_params=pltpu.CompilerParams(use_tc_tiling_on_sc=False),
    )
    def k(idx_hbm, emb1_hbm, emb2_hbm, rows_out, vals1_out,
          idx_v, rows_v, vals_v, sems):
        wid = lax.axis_index("s") * 2 + lax.axis_index("c")
        base = wid * npw
        pltpu.sync_copy(idx_hbm.at[pl.ds(base, npw)], idx_v)
        sgr = sems[0:nch]
        sgv = sems[nch:2 * nch]
        swr = sems[2 * nch:3 * nch]
        swv = sems[3 * nch:4 * nch]

        def idx_c(c):
            return idx_v.at[pl.ds(c * _CHUNK, _CHUNK)]

        def out_sl(c):
            return pl.ds(base + c * _CHUNK, _CHUNK)

        gr = [pltpu.async_copy(emb2_hbm.at[idx_c(c)], rows_v[c], sgr[c])
              for c in range(nch)]
        gv = [pltpu.async_copy(emb1_hbm.at[idx_c(c)], vals_v[c], sgv[c])
              for c in range(nch)]
        wr = [None] * nch
        wv = [None] * nch
        for c in range(nch):
            gr[c].wait()
            wr[c] = pltpu.async_copy(rows_v[c], rows_out.at[out_sl(c)],
                                     swr[c])
            gv[c].wait()
            wv[c] = pltpu.async_copy(vals_v[c], vals1_out.at[out_sl(c)],
                                     swv[c])
        for c in range(nch):
            wr[c].wait()
            wv[c].wait()

    return k


def _tc_body(e_ref, d_ref, e1_ref, fmw_ref, w1t_ref, w2t_ref, w3_ref,
             sel_ref, o_ref):
    e = e_ref[...]
    h1 = jnp.maximum(
        jnp.dot(e, w1t_ref[...], preferred_element_type=jnp.float32), 0.0)
    h2 = jnp.maximum(
        jnp.dot(h1, w2t_ref[...], preferred_element_type=jnp.float32), 0.0)
    y3 = jnp.sum(h2 * w3_ref[...], axis=1, keepdims=True)
    ssum = jnp.dot(e, sel_ref[...], preferred_element_type=jnp.float32)
    y2 = 0.5 * (jnp.sum(ssum * ssum, axis=1, keepdims=True)
                - jnp.sum(e * e, axis=1, keepdims=True))
    y1 = (jnp.sum(d_ref[...] * fmw_ref[...], axis=1, keepdims=True)
          + jnp.sum(e1_ref[...], axis=1, keepdims=True))
    z = y1 + y2 + y3
    o_ref[...] = 1.0 / (1.0 + jnp.exp(-z))


def _tc_mlp(e_flat, dense, e1, fm_w, w1t, w2t, w3, sel):
    nb = e_flat.shape[0]
    return pl.pallas_call(
        _tc_body,
        grid=(nb // _TC_BB,),
        in_specs=[
            pl.BlockSpec((_TC_BB, _S * _E), lambda i: (i, 0)),
            pl.BlockSpec((_TC_BB, 13), lambda i: (i, 0)),
            pl.BlockSpec((_TC_BB, _S), lambda i: (i, 0)),
            pl.BlockSpec((1, 13), lambda i: (0, 0)),
            pl.BlockSpec((_S * _E, 256), lambda i: (0, 0)),
            pl.BlockSpec((256, 256), lambda i: (0, 0)),
            pl.BlockSpec((1, 256), lambda i: (0, 0)),
            pl.BlockSpec((_S * _E, _E), lambda i: (0, 0)),
        ],
        out_specs=pl.BlockSpec((_TC_BB, 1), lambda i: (i, 0)),
        out_shape=jax.ShapeDtypeStruct((nb, 1), jnp.float32),
    )(e_flat, dense, e1, fm_w, w1t, w2t, w3, sel)


_SEL = np.tile(np.eye(_E, dtype=np.float32), (_S, 1))


def kernel(dense_input, sparse_input, emb1, emb2, fm_w, w1, w2, w3):
    emb1_flat = emb1.reshape(_F)
    w1t = w1.T
    w2t = w2.T
    sel = jnp.asarray(_SEL)
    outs = []
    off = 0
    for bsl in _SLICES:
        idx_sl = lax.slice_in_dim(sparse_input, off,
                                  off + bsl).astype(jnp.int32).reshape(-1)
        rows, vals1 = _sc_gather_fn(bsl)(idx_sl, emb1_flat, emb2)
        e_flat = rows.reshape(bsl, _S * _E)
        e1 = vals1.reshape(bsl, _S)
        d_sl = lax.slice_in_dim(dense_input, off, off + bsl)
        outs.append(_tc_mlp(e_flat, d_sl, e1, fm_w, w1t, w2t, w3, sel))
        off += bsl
    return jnp.concatenate(outs, axis=0)


# speedup vs baseline: 2.5653x; 1.3007x over previous
import functools

import jax
import jax.numpy as jnp
import numpy as np
from jax import lax
from jax.experimental import pallas as pl
from jax.experimental.pallas import tpu as pltpu
from jax.experimental.pallas import tpu_sc as plsc

_B = 16384
_S = 26
_E = 16
_F = 4823
_FPAD = 4824
_NW = 32
_SLICES = (6144, 6144, 4096)
_CHUNK = 1664
_TC_BB = 1024


@functools.lru_cache(maxsize=None)
def _sc_gather_fn(bsl):
    mesh = plsc.VectorSubcoreMesh(core_axis_name="c", subcore_axis_name="s")
    npw = bsl * _S // _NW
    nch = npw // _CHUNK

    @functools.partial(
        pl.kernel,
        mesh=mesh,
        out_type=[
            jax.ShapeDtypeStruct((bsl * _S, _E), jnp.float32),
            jax.ShapeDtypeStruct((bsl * _S,), jnp.float32),
        ],
        scratch_types=[
            pltpu.VMEM((npw,), jnp.int32),
            [pltpu.VMEM((_CHUNK, _E), jnp.float32)] * nch,
            [pltpu.VMEM((_CHUNK,), jnp.float32)] * nch,
            pltpu.VMEM_SHARED((_FPAD, _E), jnp.float32),
            pltpu.VMEM_SHARED((_FPAD,), jnp.float32),
            [pltpu.SemaphoreType.DMA] * (4 * nch),
        ],
        compiler_params=pltpu.CompilerParams(use_tc_tiling_on_sc=False),
    )
    def k(idx_hbm, emb1_hbm, emb2_hbm, rows_out, vals1_out,
          idx_v, rows_v, vals_v, emb2_sp, emb1_sp, sems):
        wid = lax.axis_index("s") * 2 + lax.axis_index("c")
        base = wid * npw
        @pl.when(lax.axis_index("s") == 0)
        def _stage():
            pltpu.sync_copy(emb2_hbm, emb2_sp)
            pltpu.sync_copy(emb1_hbm, emb1_sp)

        pltpu.sync_copy(idx_hbm.at[pl.ds(base, npw)], idx_v)
        plsc.subcore_barrier()
        sgr = sems[0:nch]
        sgv = sems[nch:2 * nch]
        swr = sems[2 * nch:3 * nch]
        swv = sems[3 * nch:4 * nch]

        def idx_c(c):
            return idx_v.at[pl.ds(c * _CHUNK, _CHUNK)]

        def out_sl(c):
            return pl.ds(base + c * _CHUNK, _CHUNK)

        gr = [pltpu.async_copy(emb2_sp.at[idx_c(c)], rows_v[c], sgr[c])
              for c in range(nch)]
        gv = [pltpu.async_copy(emb1_sp.at[idx_c(c)], vals_v[c], sgv[c])
              for c in range(nch)]
        wr = [None] * nch
        wv = [None] * nch
        for c in range(nch):
            gr[c].wait()
            wr[c] = pltpu.async_copy(rows_v[c], rows_out.at[out_sl(c)],
                                     swr[c])
            gv[c].wait()
            wv[c] = pltpu.async_copy(vals_v[c], vals1_out.at[out_sl(c)],
                                     swv[c])
        for c in range(nch):
            wr[c].wait()
            wv[c].wait()

    return k


def _tc_body(e_ref, d_ref, e1_ref, fmw_ref, w1t_ref, w2t_ref, w3_ref,
             sel_ref, o_ref):
    e = e_ref[...]
    h1 = jnp.maximum(
        jnp.dot(e, w1t_ref[...], preferred_element_type=jnp.float32), 0.0)
    h2 = jnp.maximum(
        jnp.dot(h1, w2t_ref[...], preferred_element_type=jnp.float32), 0.0)
    y3 = jnp.sum(h2 * w3_ref[...], axis=1, keepdims=True)
    ssum = jnp.dot(e, sel_ref[...], preferred_element_type=jnp.float32)
    y2 = 0.5 * (jnp.sum(ssum * ssum, axis=1, keepdims=True)
                - jnp.sum(e * e, axis=1, keepdims=True))
    y1 = (jnp.sum(d_ref[...] * fmw_ref[...], axis=1, keepdims=True)
          + jnp.sum(e1_ref[...], axis=1, keepdims=True))
    z = y1 + y2 + y3
    o_ref[...] = 1.0 / (1.0 + jnp.exp(-z))


def _tc_mlp(e_flat, dense, e1, fm_w, w1t, w2t, w3, sel):
    nb = e_flat.shape[0]
    return pl.pallas_call(
        _tc_body,
        grid=(nb // _TC_BB,),
        in_specs=[
            pl.BlockSpec((_TC_BB, _S * _E), lambda i: (i, 0)),
            pl.BlockSpec((_TC_BB, 13), lambda i: (i, 0)),
            pl.BlockSpec((_TC_BB, _S), lambda i: (i, 0)),
            pl.BlockSpec((1, 13), lambda i: (0, 0)),
            pl.BlockSpec((_S * _E, 256), lambda i: (0, 0)),
            pl.BlockSpec((256, 256), lambda i: (0, 0)),
            pl.BlockSpec((1, 256), lambda i: (0, 0)),
            pl.BlockSpec((_S * _E, _E), lambda i: (0, 0)),
        ],
        out_specs=pl.BlockSpec((_TC_BB, 1), lambda i: (i, 0)),
        out_shape=jax.ShapeDtypeStruct((nb, 1), jnp.float32),
    )(e_flat, dense, e1, fm_w, w1t, w2t, w3, sel)


_SEL = np.tile(np.eye(_E, dtype=np.float32), (_S, 1))


def kernel(dense_input, sparse_input, emb1, emb2, fm_w, w1, w2, w3):
    emb1_flat = jnp.pad(emb1.reshape(_F), (0, _FPAD - _F))
    emb2_pad = jnp.pad(emb2, ((0, _FPAD - _F), (0, 0)))
    w1t = w1.T
    w2t = w2.T
    sel = jnp.asarray(_SEL)
    outs = []
    off = 0
    for bsl in _SLICES:
        idx_sl = lax.slice_in_dim(sparse_input, off,
                                  off + bsl).astype(jnp.int32).reshape(-1)
        rows, vals1 = _sc_gather_fn(bsl)(idx_sl, emb1_flat, emb2_pad)
        e_flat = rows.reshape(bsl, _S * _E)
        e1 = vals1.reshape(bsl, _S)
        d_sl = lax.slice_in_dim(dense_input, off, off + bsl)
        outs.append(_tc_mlp(e_flat, d_sl, e1, fm_w, w1t, w2t, w3, sel))
        off += bsl
    return jnp.concatenate(outs, axis=0)


# speedup vs baseline: 2.7164x; 1.0589x over previous
import functools

import jax
import jax.numpy as jnp
import numpy as np
from jax import lax
from jax.experimental import pallas as pl
from jax.experimental.pallas import tpu as pltpu
from jax.experimental.pallas import tpu_sc as plsc

_B = 16384
_S = 26
_E = 16
_F = 4823
_FPAD = 4824
_NW = 32
_SLICES = (8192, 8192)
_CHUNK = 1664
_TC_BB = 1024


@functools.lru_cache(maxsize=None)
def _sc_gather_fn(bsl):
    mesh = plsc.VectorSubcoreMesh(core_axis_name="c", subcore_axis_name="s")
    npw = bsl * _S // _NW
    nch = npw // _CHUNK

    @functools.partial(
        pl.kernel,
        mesh=mesh,
        out_type=[
            jax.ShapeDtypeStruct((bsl * _S, _E), jnp.float32),
            jax.ShapeDtypeStruct((bsl * _S,), jnp.float32),
        ],
        scratch_types=[
            pltpu.VMEM((npw,), jnp.int32),
            [pltpu.VMEM((_CHUNK, _E), jnp.float32)] * nch,
            [pltpu.VMEM((_CHUNK,), jnp.float32)] * nch,
            pltpu.VMEM_SHARED((_FPAD, _E), jnp.float32),
            pltpu.VMEM_SHARED((_FPAD,), jnp.float32),
            [pltpu.SemaphoreType.DMA] * (4 * nch),
        ],
        compiler_params=pltpu.CompilerParams(use_tc_tiling_on_sc=False),
    )
    def k(idx_hbm, emb1_hbm, emb2_hbm, rows_out, vals1_out,
          idx_v, rows_v, vals_v, emb2_sp, emb1_sp, sems):
        wid = lax.axis_index("s") * 2 + lax.axis_index("c")
        base = wid * npw
        @pl.when(lax.axis_index("s") == 0)
        def _stage():
            pltpu.sync_copy(emb2_hbm, emb2_sp)
            pltpu.sync_copy(emb1_hbm, emb1_sp)

        pltpu.sync_copy(idx_hbm.at[pl.ds(base, npw)], idx_v)
        plsc.subcore_barrier()
        sgr = sems[0:nch]
        sgv = sems[nch:2 * nch]
        swr = sems[2 * nch:3 * nch]
        swv = sems[3 * nch:4 * nch]

        def idx_c(c):
            return idx_v.at[pl.ds(c * _CHUNK, _CHUNK)]

        def out_sl(c):
            return pl.ds(base + c * _CHUNK, _CHUNK)

        gr = [pltpu.async_copy(emb2_sp.at[idx_c(c)], rows_v[c], sgr[c])
              for c in range(nch)]
        gv = [pltpu.async_copy(emb1_sp.at[idx_c(c)], vals_v[c], sgv[c])
              for c in range(nch)]
        wr = [None] * nch
        wv = [None] * nch
        for c in range(nch):
            gr[c].wait()
            wr[c] = pltpu.async_copy(rows_v[c], rows_out.at[out_sl(c)],
                                     swr[c])
            gv[c].wait()
            wv[c] = pltpu.async_copy(vals_v[c], vals1_out.at[out_sl(c)],
                                     swv[c])
        for c in range(nch):
            wr[c].wait()
            wv[c].wait()

    return k


def _tc_body(e_ref, d_ref, e1_ref, fmw_ref, w1t_ref, w2t_ref, w3_ref,
             sel_ref, o_ref):
    e = e_ref[...]
    h1 = jnp.maximum(
        jnp.dot(e, w1t_ref[...], preferred_element_type=jnp.float32), 0.0)
    h2 = jnp.maximum(
        jnp.dot(h1, w2t_ref[...], preferred_element_type=jnp.float32), 0.0)
    y3 = jnp.sum(h2 * w3_ref[...], axis=1, keepdims=True)
    ssum = jnp.dot(e, sel_ref[...], preferred_element_type=jnp.float32)
    y2 = 0.5 * (jnp.sum(ssum * ssum, axis=1, keepdims=True)
                - jnp.sum(e * e, axis=1, keepdims=True))
    y1 = (jnp.sum(d_ref[...] * fmw_ref[...], axis=1, keepdims=True)
          + jnp.sum(e1_ref[...], axis=1, keepdims=True))
    z = y1 + y2 + y3
    o_ref[...] = 1.0 / (1.0 + jnp.exp(-z))


def _tc_mlp(e_flat, dense, e1, fm_w, w1t, w2t, w3, sel):
    nb = e_flat.shape[0]
    return pl.pallas_call(
        _tc_body,
        grid=(nb // _TC_BB,),
        in_specs=[
            pl.BlockSpec((_TC_BB, _S * _E), lambda i: (i, 0)),
            pl.BlockSpec((_TC_BB, 13), lambda i: (i, 0)),
            pl.BlockSpec((_TC_BB, _S), lambda i: (i, 0)),
            pl.BlockSpec((1, 13), lambda i: (0, 0)),
            pl.BlockSpec((_S * _E, 256), lambda i: (0, 0)),
            pl.BlockSpec((256, 256), lambda i: (0, 0)),
            pl.BlockSpec((1, 256), lambda i: (0, 0)),
            pl.BlockSpec((_S * _E, _E), lambda i: (0, 0)),
        ],
        out_specs=pl.BlockSpec((_TC_BB, 1), lambda i: (i, 0)),
        out_shape=jax.ShapeDtypeStruct((nb, 1), jnp.float32),
    )(e_flat, dense, e1, fm_w, w1t, w2t, w3, sel)


_SEL = np.tile(np.eye(_E, dtype=np.float32), (_S, 1))


def kernel(dense_input, sparse_input, emb1, emb2, fm_w, w1, w2, w3):
    emb1_flat = jnp.pad(emb1.reshape(_F), (0, _FPAD - _F))
    emb2_pad = jnp.pad(emb2, ((0, _FPAD - _F), (0, 0)))
    w1t = w1.T
    w2t = w2.T
    sel = jnp.asarray(_SEL)
    outs = []
    off = 0
    for bsl in _SLICES:
        idx_sl = lax.slice_in_dim(sparse_input, off,
                                  off + bsl).astype(jnp.int32).reshape(-1)
        rows, vals1 = _sc_gather_fn(bsl)(idx_sl, emb1_flat, emb2_pad)
        e_flat = rows.reshape(bsl, _S * _E)
        e1 = vals1.reshape(bsl, _S)
        d_sl = lax.slice_in_dim(dense_input, off, off + bsl)
        outs.append(_tc_mlp(e_flat, d_sl, e1, fm_w, w1t, w2t, w3, sel))
        off += bsl
    return jnp.concatenate(outs, axis=0)


# speedup vs baseline: 2.7933x; 1.0283x over previous
import functools

import jax
import jax.numpy as jnp
import numpy as np
from jax import lax
from jax.experimental import pallas as pl
from jax.experimental.pallas import tpu as pltpu
from jax.experimental.pallas import tpu_sc as plsc

_B = 16384
_S = 26
_E = 16
_F = 4823
_FPAD = 4824
_NW = 32
_SLICES = (8192, 8192)
_CHUNK = 1664
_TC_BB = 2048


@functools.lru_cache(maxsize=None)
def _sc_gather_fn(bsl):
    mesh = plsc.VectorSubcoreMesh(core_axis_name="c", subcore_axis_name="s")
    npw = bsl * _S // _NW
    nch = npw // _CHUNK

    @functools.partial(
        pl.kernel,
        mesh=mesh,
        out_type=[
            jax.ShapeDtypeStruct((bsl * _S, _E), jnp.float32),
            jax.ShapeDtypeStruct((bsl * _S,), jnp.float32),
        ],
        scratch_types=[
            pltpu.VMEM((npw,), jnp.int32),
            [pltpu.VMEM((_CHUNK, _E), jnp.float32)] * nch,
            [pltpu.VMEM((_CHUNK,), jnp.float32)] * nch,
            pltpu.VMEM_SHARED((_FPAD, _E), jnp.float32),
            pltpu.VMEM_SHARED((_FPAD,), jnp.float32),
            [pltpu.SemaphoreType.DMA] * (4 * nch),
        ],
        compiler_params=pltpu.CompilerParams(use_tc_tiling_on_sc=False),
    )
    def k(idx_hbm, emb1_hbm, emb2_hbm, rows_out, vals1_out,
          idx_v, rows_v, vals_v, emb2_sp, emb1_sp, sems):
        wid = lax.axis_index("s") * 2 + lax.axis_index("c")
        base = wid * npw
        @pl.when(lax.axis_index("s") == 0)
        def _stage():
            pltpu.sync_copy(emb2_hbm, emb2_sp)
            pltpu.sync_copy(emb1_hbm, emb1_sp)

        pltpu.sync_copy(idx_hbm.at[pl.ds(base, npw)], idx_v)
        plsc.subcore_barrier()
        sgr = sems[0:nch]
        sgv = sems[nch:2 * nch]
        swr = sems[2 * nch:3 * nch]
        swv = sems[3 * nch:4 * nch]

        def idx_c(c):
            return idx_v.at[pl.ds(c * _CHUNK, _CHUNK)]

        def out_sl(c):
            return pl.ds(base + c * _CHUNK, _CHUNK)

        gr = [pltpu.async_copy(emb2_sp.at[idx_c(c)], rows_v[c], sgr[c])
              for c in range(nch)]
        gv = [pltpu.async_copy(emb1_sp.at[idx_c(c)], vals_v[c], sgv[c])
              for c in range(nch)]
        wr = [None] * nch
        wv = [None] * nch
        for c in range(nch):
            gr[c].wait()
            wr[c] = pltpu.async_copy(rows_v[c], rows_out.at[out_sl(c)],
                                     swr[c])
            gv[c].wait()
            wv[c] = pltpu.async_copy(vals_v[c], vals1_out.at[out_sl(c)],
                                     swv[c])
        for c in range(nch):
            wr[c].wait()
            wv[c].wait()

    return k


def _tc_body(e_ref, d_ref, e1_ref, fmw_ref, w1t_ref, w2t_ref, w3_ref,
             sel_ref, o_ref):
    e = e_ref[...]
    h1 = jnp.maximum(
        jnp.dot(e, w1t_ref[...], preferred_element_type=jnp.float32), 0.0)
    h2 = jnp.maximum(
        jnp.dot(h1, w2t_ref[...], preferred_element_type=jnp.float32), 0.0)
    y3 = jnp.sum(h2 * w3_ref[...], axis=1, keepdims=True)
    ssum = jnp.dot(e, sel_ref[...], preferred_element_type=jnp.float32)
    y2 = 0.5 * (jnp.sum(ssum * ssum, axis=1, keepdims=True)
                - jnp.sum(e * e, axis=1, keepdims=True))
    y1 = (jnp.sum(d_ref[...] * fmw_ref[...], axis=1, keepdims=True)
          + jnp.sum(e1_ref[...], axis=1, keepdims=True))
    z = y1 + y2 + y3
    o_ref[...] = 1.0 / (1.0 + jnp.exp(-z))


def _tc_mlp(e_flat, dense, e1, fm_w, w1t, w2t, w3, sel):
    nb = e_flat.shape[0]
    return pl.pallas_call(
        _tc_body,
        grid=(nb // _TC_BB,),
        in_specs=[
            pl.BlockSpec((_TC_BB, _S * _E), lambda i: (i, 0)),
            pl.BlockSpec((_TC_BB, 13), lambda i: (i, 0)),
            pl.BlockSpec((_TC_BB, _S), lambda i: (i, 0)),
            pl.BlockSpec((1, 13), lambda i: (0, 0)),
            pl.BlockSpec((_S * _E, 256), lambda i: (0, 0)),
            pl.BlockSpec((256, 256), lambda i: (0, 0)),
            pl.BlockSpec((1, 256), lambda i: (0, 0)),
            pl.BlockSpec((_S * _E, _E), lambda i: (0, 0)),
        ],
        out_specs=pl.BlockSpec((_TC_BB, 1), lambda i: (i, 0)),
        out_shape=jax.ShapeDtypeStruct((nb, 1), jnp.float32),
    )(e_flat, dense, e1, fm_w, w1t, w2t, w3, sel)


_SEL = np.tile(np.eye(_E, dtype=np.float32), (_S, 1))


def kernel(dense_input, sparse_input, emb1, emb2, fm_w, w1, w2, w3):
    emb1_flat = jnp.pad(emb1.reshape(_F), (0, _FPAD - _F))
    emb2_pad = jnp.pad(emb2, ((0, _FPAD - _F), (0, 0)))
    w1t = w1.T
    w2t = w2.T
    sel = jnp.asarray(_SEL)
    outs = []
    off = 0
    for bsl in _SLICES:
        idx_sl = lax.slice_in_dim(sparse_input, off,
                                  off + bsl).astype(jnp.int32).reshape(-1)
        rows, vals1 = _sc_gather_fn(bsl)(idx_sl, emb1_flat, emb2_pad)
        e_flat = rows.reshape(bsl, _S * _E)
        e1 = vals1.reshape(bsl, _S)
        d_sl = lax.slice_in_dim(dense_input, off, off + bsl)
        outs.append(_tc_mlp(e_flat, d_sl, e1, fm_w, w1t, w2t, w3, sel))
        off += bsl
    return jnp.concatenate(outs, axis=0)
